# Initial kernel scaffold; baseline (speedup 1.0000x reference)
import jax
import jax.numpy as jnp
from jax.experimental import pallas as pl


def kernel(x, token_table, pos_table):
    raise NotImplementedError("write your pallas kernel here")


# speedup vs baseline: 2.9706x; 2.9706x over previous
import functools

import jax
import jax.numpy as jnp
from jax import lax
from jax.experimental import pallas as pl
from jax.experimental.pallas import tpu as pltpu
from jax.experimental.pallas import tpu_sc as plsc

L_SEQ = 200
D = 64
LANES = 16
SCALE = 8.0

NC = 2
NS = 16
NW = NC * NS

CHUNK = 800
IDX_MINOR = 100
IDX_ROWS = CHUNK // IDX_MINOR


def _make_sc_kernel(n_rows):
    rows_per_w = n_rows // NW
    n_chunks = rows_per_w // CHUNK
    assert rows_per_w % CHUNK == 0 and CHUNK % L_SEQ == 0
    mesh = plsc.VectorSubcoreMesh(core_axis_name="c", subcore_axis_name="s")

    @functools.partial(
        pl.kernel,
        mesh=mesh,
        out_type=jax.ShapeDtypeStruct((n_rows, D), jnp.float32),
        scratch_types=[
            pltpu.VMEM((IDX_ROWS, IDX_MINOR), jnp.int32),
            pltpu.VMEM((CHUNK, D), jnp.float32),
            pltpu.VMEM((L_SEQ, D), jnp.float32),
            pltpu.SemaphoreType.DMA,
        ],
        compiler_params=pltpu.CompilerParams(use_tc_tiling_on_sc=False),
    )
    def k(idx_hbm, tok_hbm, pos_hbm, out_hbm, idx_v, rows_v, pos_v, sem):
        wid = lax.axis_index("s") * NC + lax.axis_index("c")
        base = wid * rows_per_w
        pltpu.sync_copy(pos_hbm, pos_v)
        for ch in range(n_chunks):
            start = base + ch * CHUNK
            irow = pl.multiple_of(start // IDX_MINOR, 8)
            pltpu.sync_copy(idx_hbm.at[pl.ds(irow, IDX_ROWS)], idx_v)
            copies = [
                pltpu.async_copy(
                    tok_hbm.at[idx_v.at[j]],
                    rows_v.at[pl.ds(j * IDX_MINOR, IDX_MINOR)],
                    sem,
                )
                for j in range(IDX_ROWS)
            ]
            for cp in copies:
                cp.wait()

            def body(r, carry):
                for c in range(D // LANES):
                    pv = pos_v[r, pl.ds(c * LANES, LANES)]
                    for p in range(CHUNK // L_SEQ):
                        row = p * L_SEQ + r
                        sl = pl.ds(c * LANES, LANES)
                        rows_v[row, sl] = rows_v[row, sl] * SCALE + pv
                return carry

            lax.fori_loop(0, L_SEQ, body, 0)
            pltpu.sync_copy(rows_v, out_hbm.at[pl.ds(start, CHUNK)])

    return k


def kernel(x, token_table, pos_table):
    b, l = x.shape
    d = token_table.shape[1]
    idx = x.reshape(b * l // IDX_MINOR, IDX_MINOR).astype(jnp.int32)
    out = _make_sc_kernel(b * l)(idx, token_table, pos_table)
    return out.reshape(b, l, d)


# speedup vs baseline: 3.1021x; 1.0443x over previous
import functools

import jax
import jax.numpy as jnp
from jax import lax
from jax.experimental import pallas as pl
from jax.experimental.pallas import tpu as pltpu
from jax.experimental.pallas import tpu_sc as plsc

L_SEQ = 200
D = 64
LANES = 16
SCALE = 8.0

NC = 2
NS = 16
NW = NC * NS

CHUNK = 800
IDX_MINOR = 100
IDX_ROWS = CHUNK // IDX_MINOR


def _make_sc_kernel(n_rows):
    rows_per_w = n_rows // NW
    n_chunks = rows_per_w // CHUNK
    assert rows_per_w % CHUNK == 0 and CHUNK % L_SEQ == 0
    mesh = plsc.VectorSubcoreMesh(core_axis_name="c", subcore_axis_name="s")

    @functools.partial(
        pl.kernel,
        mesh=mesh,
        out_type=jax.ShapeDtypeStruct((n_rows, D), jnp.float32),
        scratch_types=[
            pltpu.VMEM((IDX_ROWS, IDX_MINOR), jnp.int32),
            pltpu.VMEM((IDX_ROWS, IDX_MINOR), jnp.int32),
            pltpu.VMEM((CHUNK, D), jnp.float32),
            pltpu.VMEM((CHUNK, D), jnp.float32),
            pltpu.VMEM((L_SEQ, D), jnp.float32),
            pltpu.SemaphoreType.DMA,
            pltpu.SemaphoreType.DMA,
            pltpu.SemaphoreType.DMA,
            pltpu.SemaphoreType.DMA,
        ],
        compiler_params=pltpu.CompilerParams(use_tc_tiling_on_sc=False),
    )
    def k(idx_hbm, tok_hbm, pos_hbm, out_hbm,
          idx0, idx1, rows0, rows1, pos_v, sg0, sg1, so0, so1):
        wid = lax.axis_index("s") * NC + lax.axis_index("c")
        base = wid * rows_per_w
        idx_bufs, row_bufs = (idx0, idx1), (rows0, rows1)
        sem_g, sem_o = (sg0, sg1), (so0, so1)

        pltpu.sync_copy(pos_hbm, pos_v)

        def stage_gather(ch, b):
            start = base + ch * CHUNK
            irow = pl.multiple_of(start // IDX_MINOR, 8)
            pltpu.sync_copy(idx_hbm.at[pl.ds(irow, IDX_ROWS)], idx_bufs[b])
            return [
                pltpu.async_copy(
                    tok_hbm.at[idx_bufs[b].at[j]],
                    row_bufs[b].at[pl.ds(j * IDX_MINOR, IDX_MINOR)],
                    sem_g[b],
                )
                for j in range(IDX_ROWS)
            ]

        pending_g = {0: stage_gather(0, 0)}
        pending_o = {}
        for ch in range(n_chunks):
            b = ch % 2
            if ch + 1 < n_chunks:
                if (ch - 1) in pending_o:
                    pending_o.pop(ch - 1).wait()
                pending_g[ch + 1] = stage_gather(ch + 1, 1 - b)
            for cp in pending_g.pop(ch):
                cp.wait()

            rows_b = row_bufs[b]

            @plsc.parallel_loop(0, L_SEQ, unroll=4)
            def body(r):
                for c in range(D // LANES):
                    sl = pl.ds(c * LANES, LANES)
                    pv = pos_v[r, sl]
                    for p in range(CHUNK // L_SEQ):
                        row = p * L_SEQ + r
                        rows_b[row, sl] = rows_b[row, sl] * SCALE + pv

            start = base + ch * CHUNK
            pending_o[ch] = pltpu.async_copy(
                rows_b, out_hbm.at[pl.ds(start, CHUNK)], sem_o[b])
        for cp in pending_o.values():
            cp.wait()

    return k


def kernel(x, token_table, pos_table):
    b, l = x.shape
    d = token_table.shape[1]
    idx = x.reshape(b * l // IDX_MINOR, IDX_MINOR).astype(jnp.int32)
    out = _make_sc_kernel(b * l)(idx, token_table, pos_table)
    return out.reshape(b, l, d)
